# baseline (speedup 1.0000x reference)
import functools

import jax
import jax.numpy as jnp
from jax.experimental import pallas as pl
from jax.experimental.pallas import tpu as pltpu

_Q = 1024
_D = 128
_BK = 10240
_CH = 512
_NCH = _BK // _CH
_SLAB = 8
_NSL = _CH // _SLAB
_BIG = 2**30


def _nn_kernel(num_k, total_k, pts_ref, grid_ref, p2_ref, g2_ref,
               dist_ref, idx_ref, min_sc, idx_sc):
    k = pl.program_id(0)

    @pl.when(k == 0)
    def _init():
        min_sc[...] = jnp.full((1, _Q), jnp.inf, jnp.float32)
        idx_sc[...] = jnp.zeros((1, _Q), jnp.int32)

    g = grid_ref[...]
    g2 = g2_ref[...]
    row_ids = jax.lax.broadcasted_iota(jnp.int32, (_BK, 1), 0)
    p2 = jnp.where(row_ids < (total_k - k * _BK), p2_ref[...], jnp.inf)

    def mm(c):
        pc = jax.lax.slice(pts_ref[...], (c * _CH, 0), ((c + 1) * _CH, _D))
        dots = jax.lax.dot_general(
            pc * (-2.0), g, (((1,), (1,)), ((), ())),
            preferred_element_type=jnp.float32)
        p2c = jax.lax.slice(p2, (c * _CH, 0), ((c + 1) * _CH, 1))
        return (g2 + dots) + p2c

    def red(c, d2c, carry):
        run_min, run_r = carry
        for r in range(_NSL):
            d2 = jax.lax.slice(d2c, (r * _SLAB, 0), ((r + 1) * _SLAB, _Q))
            take = d2 < run_min
            run_min = jnp.where(take, d2, run_min)
            run_r = jnp.where(take, c * _NSL + r, run_r)
        return run_min, run_r

    carry = (jnp.full((_SLAB, _Q), jnp.inf, jnp.float32),
             jnp.zeros((_SLAB, _Q), jnp.int32))
    d2_prev = mm(0)
    for c in range(1, _NCH):
        d2c = mm(c)
        carry = red(c - 1, d2_prev, carry)
        d2_prev = d2c
    run_min, run_r = red(_NCH - 1, d2_prev, carry)

    m = jnp.min(run_min, axis=0, keepdims=True)
    iota_s = jax.lax.broadcasted_iota(jnp.int32, (_SLAB, _Q), 0)
    br = run_r * _SLAB + iota_s
    cand = jnp.where(run_min == m, br, _BIG)
    j = jnp.min(cand, axis=0, keepdims=True)

    cur_min = min_sc[...]
    take2 = m < cur_min
    min_sc[...] = jnp.where(take2, m, cur_min)
    idx_sc[...] = jnp.where(take2, j + k * _BK, idx_sc[...])

    @pl.when(k == num_k - 1)
    def _finalize():
        dist_ref[...] = jnp.sqrt(jnp.maximum(min_sc[...], 0.0))
        idx_ref[...] = idx_sc[...]


@jax.jit
def kernel(points, grid):
    total_k, d = points.shape
    q = grid.shape[0]
    num_k = pl.cdiv(total_k, _BK)
    p2 = jnp.sum(points * points, axis=1).reshape(total_k, 1)
    g2 = jnp.sum(grid * grid, axis=1).reshape(1, q)
    dist, idx = pl.pallas_call(
        functools.partial(_nn_kernel, num_k, total_k),
        grid=(num_k,),
        in_specs=[
            pl.BlockSpec((_BK, _D), lambda k: (k, 0)),
            pl.BlockSpec((_Q, _D), lambda k: (0, 0)),
            pl.BlockSpec((_BK, 1), lambda k: (k, 0)),
            pl.BlockSpec((1, _Q), lambda k: (0, 0)),
        ],
        out_specs=[
            pl.BlockSpec((1, _Q), lambda k: (0, 0)),
            pl.BlockSpec((1, _Q), lambda k: (0, 0)),
        ],
        out_shape=[
            jax.ShapeDtypeStruct((1, q), jnp.float32),
            jax.ShapeDtypeStruct((1, q), jnp.int32),
        ],
        scratch_shapes=[
            pltpu.VMEM((1, _Q), jnp.float32),
            pltpu.VMEM((1, _Q), jnp.int32),
        ],
        compiler_params=pltpu.CompilerParams(
            dimension_semantics=("arbitrary",)),
    )(points, grid, p2, g2)
    return dist.reshape(q), idx.reshape(q)


# speedup vs baseline: 1.3149x; 1.3149x over previous
import functools

import jax
import jax.numpy as jnp
from jax.experimental import pallas as pl
from jax.experimental.pallas import tpu as pltpu

_Q = 1024
_D = 128
_BK = 10240
_CH = 512
_NCH = _BK // _CH
_SLAB = 8
_NSL = _CH // _SLAB
_BIG = 2**30


def _nn_kernel(num_k, total_k, pts_ref, grid_ref, p2_ref, g2_ref,
               dist_ref, idx_ref, min_sc, idx_sc):
    k = pl.program_id(0)

    @pl.when(k == 0)
    def _init():
        min_sc[...] = jnp.full((1, _Q), jnp.inf, jnp.float32)
        idx_sc[...] = jnp.zeros((1, _Q), jnp.int32)

    g = grid_ref[...]
    g2 = g2_ref[...]
    p2t = p2_ref[...]

    def mm(c):
        pc = jax.lax.slice(pts_ref[...], (c * _CH, 0), ((c + 1) * _CH, _D))
        dots = jax.lax.dot_general(
            pc * (-2.0), g, (((1,), (1,)), ((), ())),
            preferred_element_type=jnp.float32)
        return g2 + dots

    def red(c, d2c, carry):
        run_min, run_r = carry
        for r in range(_NSL):
            sl = jax.lax.slice(d2c, (r * _SLAB, 0), ((r + 1) * _SLAB, _Q))
            rg = (c * _CH) // _SLAB + r
            p2sl = jax.lax.slice(p2t, (0, rg), (_SLAB, rg + 1))
            d2 = sl + p2sl
            take = d2 < run_min
            run_min = jnp.where(take, d2, run_min)
            run_r = jnp.where(take, c * _NSL + r, run_r)
        return run_min, run_r

    carry = (jnp.full((_SLAB, _Q), jnp.inf, jnp.float32),
             jnp.zeros((_SLAB, _Q), jnp.int32))
    d2_prev = mm(0)
    for c in range(1, _NCH):
        d2c = mm(c)
        carry = red(c - 1, d2_prev, carry)
        d2_prev = d2c
    run_min, run_r = red(_NCH - 1, d2_prev, carry)

    m = jnp.min(run_min, axis=0, keepdims=True)
    iota_s = jax.lax.broadcasted_iota(jnp.int32, (_SLAB, _Q), 0)
    br = run_r * _SLAB + iota_s
    cand = jnp.where(run_min == m, br, _BIG)
    j = jnp.min(cand, axis=0, keepdims=True)

    cur_min = min_sc[...]
    take2 = m < cur_min
    min_sc[...] = jnp.where(take2, m, cur_min)
    idx_sc[...] = jnp.where(take2, j + k * _BK, idx_sc[...])

    @pl.when(k == num_k - 1)
    def _finalize():
        dist_ref[...] = jnp.sqrt(jnp.maximum(min_sc[...], 0.0))
        idx_ref[...] = idx_sc[...]


@jax.jit
def kernel(points, grid):
    total_k, d = points.shape
    q = grid.shape[0]
    num_k = pl.cdiv(total_k, _BK)
    pad = num_k * _BK - total_k
    p2 = jnp.sum(points * points, axis=1)
    p2 = jnp.concatenate([p2, jnp.full((pad,), jnp.inf, jnp.float32)])
    p2 = p2.reshape(-1, _SLAB).T
    g2 = jnp.sum(grid * grid, axis=1).reshape(1, q)
    dist, idx = pl.pallas_call(
        functools.partial(_nn_kernel, num_k, total_k),
        grid=(num_k,),
        in_specs=[
            pl.BlockSpec((_BK, _D), lambda k: (k, 0)),
            pl.BlockSpec((_Q, _D), lambda k: (0, 0)),
            pl.BlockSpec((_SLAB, _BK // _SLAB), lambda k: (0, k)),
            pl.BlockSpec((1, _Q), lambda k: (0, 0)),
        ],
        out_specs=[
            pl.BlockSpec((1, _Q), lambda k: (0, 0)),
            pl.BlockSpec((1, _Q), lambda k: (0, 0)),
        ],
        out_shape=[
            jax.ShapeDtypeStruct((1, q), jnp.float32),
            jax.ShapeDtypeStruct((1, q), jnp.int32),
        ],
        scratch_shapes=[
            pltpu.VMEM((1, _Q), jnp.float32),
            pltpu.VMEM((1, _Q), jnp.int32),
        ],
        compiler_params=pltpu.CompilerParams(
            dimension_semantics=("arbitrary",)),
    )(points, grid, p2, g2)
    return dist.reshape(q), idx.reshape(q)
